# baseline (device time: 104428 ns/iter reference)
import jax
import jax.numpy as jnp
from jax import lax
from jax.experimental import pallas as pl
from jax.experimental.pallas import tpu as pltpu

P = 8
_ENABLE_SENDS = True

_TOFFS = [1, 2, 3, 4, 5, 6, 7, 0]
_RECV_AFTER_PAIR = {2: [1, 2], 3: [3, 4]}
_RECV_TAIL = [5, 6, 7]


def _body(x_ref, sx_ref, sw_ref, w_hbm, out_ref,
          w_vmem, y_pair, y_send, y_recv,
          copy_sems, send_sems, recv_sems):
    m_loc, k = x_ref.shape
    nb = out_ref.shape[1]
    kb = w_vmem.shape[1]
    n_k = k // kb
    n_chunks = (P // 2) * n_k
    me = lax.axis_index("i")
    s = sx_ref[0] * sw_ref[0]

    barrier_sem = pltpu.get_barrier_semaphore()
    for t in range(1, P):
        pl.semaphore_signal(
            barrier_sem, inc=1,
            device_id=(lax.rem(me + t, P),),
            device_id_type=pl.DeviceIdType.MESH,
        )
    pl.semaphore_wait(barrier_sem, P - 1)

    def start_w_dma(c):
        slot = c % 2
        p, kk = c // n_k, c % n_k
        cps = []
        for j in range(2):
            d = lax.rem(me + _TOFFS[2 * p + j], P)
            cp = pltpu.make_async_copy(
                w_hbm.at[pl.ds(kk * kb, kb), pl.ds(d * nb, nb)],
                w_vmem.at[slot, :, pl.ds(j * nb, nb)],
                copy_sems.at[slot, j],
            )
            cp.start()
            cps.append(cp)
        return cps

    def wait_recv_slot(t):
        origin = lax.rem(me - t + P, P)
        recv = pltpu.make_async_remote_copy(
            src_ref=y_send.at[t - 1],
            dst_ref=y_recv.at[t - 1],
            send_sem=send_sems.at[t],
            recv_sem=recv_sems.at[t],
            device_id=(me,),
            device_id_type=pl.DeviceIdType.MESH,
        )
        recv.wait_recv()
        out_ref[pl.ds(origin * m_loc, m_loc), :] = (
            y_recv[t - 1, :, :].astype(jnp.float32))

    pending = [start_w_dma(0), start_w_dma(1)]

    rdmas = []
    for c in range(n_chunks):
        p, kk = c // n_k, c % n_k
        for cp in pending[c % 2]:
            cp.wait()
        acc = jnp.dot(x_ref[:, pl.ds(kk * kb, kb)], w_vmem[c % 2, :, :],
                      preferred_element_type=jnp.float32)
        if kk == 0:
            y_pair[:, :] = acc
        else:
            y_pair[:, :] = y_pair[:, :] + acc
        if c + 2 < n_chunks:
            pending[c % 2] = start_w_dma(c + 2)

        if kk == n_k - 1:
            for j in range(2):
                t = _TOFFS[2 * p + j]
                y = jnp.maximum(y_pair[:, pl.ds(j * nb, nb)] * s, 0.0)
                if t == 0:
                    out_ref[pl.ds(me * m_loc, m_loc), :] = y
                elif _ENABLE_SENDS:
                    y_send[t - 1, :, :] = y.astype(jnp.bfloat16)
                    rdma = pltpu.make_async_remote_copy(
                        src_ref=y_send.at[t - 1],
                        dst_ref=y_recv.at[t - 1],
                        send_sem=send_sems.at[t],
                        recv_sem=recv_sems.at[t],
                        device_id=(lax.rem(me + t, P),),
                        device_id_type=pl.DeviceIdType.MESH,
                    )
                    rdma.start()
                    rdmas.append(rdma)
                else:
                    out_ref[pl.ds(0, m_loc), :] = y
            if _ENABLE_SENDS:
                for t in _RECV_AFTER_PAIR.get(p, []):
                    wait_recv_slot(t)

    if _ENABLE_SENDS:
        for t in _RECV_TAIL:
            wait_recv_slot(t)

    for rdma in rdmas:
        rdma.wait_send()


def kernel(x, w_mat, scale_x, scale_w):
    m_loc, k = x.shape
    n = w_mat.shape[1]
    nb = n // P
    kb = k // 4
    return pl.pallas_call(
        _body,
        out_shape=jax.ShapeDtypeStruct((P * m_loc, nb), jnp.float32),
        in_specs=[
            pl.BlockSpec(memory_space=pltpu.VMEM),
            pl.BlockSpec(memory_space=pltpu.SMEM),
            pl.BlockSpec(memory_space=pltpu.SMEM),
            pl.BlockSpec(memory_space=pltpu.MemorySpace.HBM),
        ],
        out_specs=pl.BlockSpec(memory_space=pltpu.VMEM),
        scratch_shapes=[
            pltpu.VMEM((2, kb, 2 * nb), jnp.float32),
            pltpu.VMEM((m_loc, 2 * nb), jnp.float32),
            pltpu.VMEM((P - 1, m_loc, nb), jnp.bfloat16),
            pltpu.VMEM((P - 1, m_loc, nb), jnp.bfloat16),
            pltpu.SemaphoreType.DMA((2, 2)),
            pltpu.SemaphoreType.DMA((P,)),
            pltpu.SemaphoreType.DMA((P,)),
        ],
        compiler_params=pltpu.CompilerParams(
            vmem_limit_bytes=62 * 1024 * 1024,
            collective_id=0),
    )(x, scale_x, scale_w, w_mat)
